# Initial kernel scaffold; baseline (speedup 1.0000x reference)
import jax
import jax.numpy as jnp
from jax.experimental import pallas as pl


def kernel(title_table, genre_table, movie_title, movie_genres):
    raise NotImplementedError("write your pallas kernel here")


# speedup vs baseline: 1.7395x; 1.7395x over previous
import functools

import jax
import jax.numpy as jnp
from jax import lax
from jax.experimental import pallas as pl
from jax.experimental.pallas import tpu as pltpu
from jax.experimental.pallas import tpu_sc as plsc

B = 16384
EMBED = 32
N_GENRES = 8
NC = 2
NS = 16
NW = NC * NS
BPW = B // NW
IDX_CHUNK = 128
T_CHUNKS = BPW // IDX_CHUNK
G_CHUNKS = BPW * N_GENRES // IDX_CHUNK
G_BATCH = IDX_CHUNK // N_GENRES


def _body(title_tab, genre_tab, tidx_hbm, gidx_hbm, out_hbm,
          tidx_v, gidx_v, trows_v, grows_v, outbuf_v, tsem, gsem):
    wid = lax.axis_index("s") * NC + lax.axis_index("c")
    base = wid * BPW

    pltpu.sync_copy(tidx_hbm.at[pl.ds(wid * T_CHUNKS, T_CHUNKS)], tidx_v)
    pltpu.sync_copy(gidx_hbm.at[pl.ds(wid * G_CHUNKS, G_CHUNKS)], gidx_v)

    tcopies = []
    for j in range(T_CHUNKS):
        tcopies.append(pltpu.async_copy(
            title_tab.at[tidx_v.at[j]],
            trows_v.at[pl.ds(j * IDX_CHUNK, IDX_CHUNK)], tsem))

    for c in range(G_CHUNKS):
        pltpu.async_copy(genre_tab.at[gidx_v.at[c]], grows_v, gsem).wait()

        def reduce_body(b, _, c=c):
            for k in range(EMBED // 16):
                acc = grows_v[b * N_GENRES, pl.ds(k * 16, 16)]
                for g in range(1, N_GENRES):
                    acc = acc + grows_v[b * N_GENRES + g, pl.ds(k * 16, 16)]
                outbuf_v[c * G_BATCH + b, pl.ds(EMBED + k * 16, 16)] = acc * 0.125
            return _
        lax.fori_loop(0, G_BATCH, reduce_body, None)

    for cp in tcopies:
        cp.wait()

    def merge_body(b, _):
        for k in range(EMBED // 16):
            outbuf_v[b, pl.ds(k * 16, 16)] = trows_v[b, pl.ds(k * 16, 16)]
        return _
    lax.fori_loop(0, BPW, merge_body, None)

    pltpu.sync_copy(outbuf_v, out_hbm.at[pl.ds(base, BPW)])


@functools.partial(jax.jit, static_argnames=())
def _run(title_table, genre_table, tidx, gidx):
    mesh = plsc.VectorSubcoreMesh(core_axis_name="c", subcore_axis_name="s",
                                  num_cores=NC, num_subcores=NS)
    return pl.kernel(
        _body,
        out_type=jax.ShapeDtypeStruct((B, 2 * EMBED), jnp.float32),
        mesh=mesh,
        scratch_types=[
            pltpu.VMEM((T_CHUNKS, IDX_CHUNK), jnp.int32),
            pltpu.VMEM((G_CHUNKS, IDX_CHUNK), jnp.int32),
            pltpu.VMEM((BPW, EMBED), jnp.float32),
            pltpu.VMEM((IDX_CHUNK, EMBED), jnp.float32),
            pltpu.VMEM((BPW, 2 * EMBED), jnp.float32),
            pltpu.SemaphoreType.DMA,
            pltpu.SemaphoreType.DMA,
        ],
        compiler_params=pltpu.CompilerParams(use_tc_tiling_on_sc=False),
    )(title_table, genre_table, tidx, gidx)


def kernel(title_table, genre_table, movie_title, movie_genres):
    tidx = movie_title.astype(jnp.int32).reshape(NW * T_CHUNKS, IDX_CHUNK)
    gidx = movie_genres.astype(jnp.int32).reshape(NW * G_CHUNKS, IDX_CHUNK)
    return _run(title_table, genre_table, tidx, gidx)
